# baseline (speedup 1.0000x reference)
import jax
import jax.numpy as jnp
from jax.experimental import pallas as pl
from jax.experimental.pallas import tpu as pltpu

_N = 4096
_NFEAT = 128
_F0, _F1, _F2 = 64, 68, 72
_SUMF = _F0 + _F1 + _F2
_H4 = _F0 * 2 + 4
_H5 = _F0 * 2
_W2C = _H5 + _H4
_FINC = _SUMF + _H4
_BM = 128
_NB = _N // _BM


def _phase0(adj_ref, s1_ref, bc_ref, lrB_ref, lr_out, abf_ref, row0):
    abf_blk = adj_ref[...].astype(jnp.bfloat16)
    abf_ref[pl.ds(row0, _BM), :] = abf_blk
    z = jnp.dot(abf_blk, s1_ref[...],
                preferred_element_type=jnp.float32) + bc_ref[...]
    sig = jax.nn.sigmoid(z)
    lane = jax.lax.broadcasted_iota(jnp.int32, sig.shape, 1)
    sec_mask = (lane >= _F0) & (lane < _F0 + _F1)
    msec = jnp.sum(jnp.where(sec_mask, sig, 0.0), axis=1,
                   keepdims=True) * (1.0 / _F1)
    lr_blk = jnp.where(lane < _F0 + _F1, sig, msec * sig)
    lrB_ref[pl.ds(row0, _BM), :] = lr_blk.astype(jnp.bfloat16)
    lr_out[...] = lr_blk


def _phase1(s2_ref, b45_ref, wm_ref, bm_ref, lrB_ref,
            fin_out, fiv_out, mlp_out, abf_ref, row0):
    ablk = abf_ref[pl.ds(row0, _BM), :]
    z = jnp.dot(ablk, s2_ref[...],
                preferred_element_type=jnp.float32) + b45_ref[...]
    fiv = z[:, :_H5]
    fou = z[:, _H5:]
    mlp = jnp.dot(fiv.astype(jnp.bfloat16), wm_ref[...],
                  preferred_element_type=jnp.float32) + bm_ref[...]
    mlp = jnp.where(mlp >= 0.0, mlp, 0.01 * mlp)
    f3 = (mlp + fou) * 0.5
    lr = lrB_ref[pl.ds(row0, _BM), :].astype(jnp.float32)
    low = jnp.mean(lr, axis=1, keepdims=True) * lr + lr
    fin_out[...] = jnp.concatenate([low, f3], axis=1)
    fiv_out[...] = fiv
    mlp_out[...] = mlp


def _body(x_ref, y_ref, adj1_ref, adj2_ref, wc_ref, bc_ref, w45_ref,
          b45_ref, wm_ref, bm_ref,
          xlr_out, xfin_out, xfiv_out, xmlp_out,
          ylr_out, yfin_out, yfiv_out, ymlp_out,
          s1_ref, s2_ref, lrx_ref, lry_ref, abf_ref):
    s = pl.program_id(0)
    i = pl.program_id(1)
    row0 = pl.multiple_of(i * _BM, _BM)

    @pl.when((s == 0) & (i == 0))
    def _():
        s1_ref[...] = jnp.dot(x_ref[...], wc_ref[...],
                              preferred_element_type=jnp.float32
                              ).astype(jnp.bfloat16)

    @pl.when((s == 1) & (i == 0))
    def _():
        s2_ref[...] = jnp.dot(lrx_ref[...], w45_ref[...],
                              preferred_element_type=jnp.float32
                              ).astype(jnp.bfloat16)
        s1_ref[...] = jnp.dot(y_ref[...], wc_ref[...],
                              preferred_element_type=jnp.float32
                              ).astype(jnp.bfloat16)

    @pl.when((s == 2) & (i == 0))
    def _():
        s2_ref[...] = jnp.dot(lry_ref[...], w45_ref[...],
                              preferred_element_type=jnp.float32
                              ).astype(jnp.bfloat16)

    @pl.when(s == 1)
    def _():
        _phase1(s2_ref, b45_ref, wm_ref, bm_ref, lrx_ref,
                xfin_out, xfiv_out, xmlp_out, abf_ref, row0)

    @pl.when(s == 2)
    def _():
        _phase1(s2_ref, b45_ref, wm_ref, bm_ref, lry_ref,
                yfin_out, yfiv_out, ymlp_out, abf_ref, row0)

    @pl.when(s == 0)
    def _():
        _phase0(adj1_ref, s1_ref, bc_ref, lrx_ref, xlr_out, abf_ref, row0)

    @pl.when(s == 1)
    def _():
        _phase0(adj2_ref, s1_ref, bc_ref, lry_ref, ylr_out, abf_ref, row0)


def kernel(x, adj1, y, adj2, W1, b1, W2, b2, W3, b3, W4, b4, W5, b5, Wm, bm):
    wc = jnp.concatenate([W1, W2, W3], axis=1).astype(jnp.bfloat16)
    bc = jnp.concatenate([b1, b2, b3]).reshape(1, _SUMF)
    w45 = jnp.concatenate([W5, W4], axis=1).astype(jnp.bfloat16)
    b45 = jnp.concatenate([b5, b4]).reshape(1, _W2C)
    wmT = Wm.T.astype(jnp.bfloat16)
    bm2 = bm.reshape(1, _H4)
    xbf = x.astype(jnp.bfloat16)
    ybf = y.astype(jnp.bfloat16)

    last = _NB - 1
    outs = pl.pallas_call(
        _body,
        grid=(3, _NB),
        in_specs=[
            pl.BlockSpec((_N, _NFEAT), lambda s, i: (0, 0)),
            pl.BlockSpec((_N, _NFEAT), lambda s, i: (0, 0)),
            pl.BlockSpec((_BM, _N),
                         lambda s, i: (jnp.where(s == 0, i, last), 0)),
            pl.BlockSpec((_BM, _N),
                         lambda s, i: (jnp.where(s == 1, i,
                                                 jnp.where(s == 0, 0, last)), 0)),
            pl.BlockSpec((_NFEAT, _SUMF), lambda s, i: (0, 0)),
            pl.BlockSpec((1, _SUMF), lambda s, i: (0, 0)),
            pl.BlockSpec((_SUMF, _W2C), lambda s, i: (0, 0)),
            pl.BlockSpec((1, _W2C), lambda s, i: (0, 0)),
            pl.BlockSpec((_H5, _H4), lambda s, i: (0, 0)),
            pl.BlockSpec((1, _H4), lambda s, i: (0, 0)),
        ],
        out_specs=[
            pl.BlockSpec((_BM, _SUMF),
                         lambda s, i: (jnp.where(s == 0, i, last), 0)),
            pl.BlockSpec((_BM, _FINC),
                         lambda s, i: (jnp.where(s == 1, i,
                                                 jnp.where(s == 0, 0, last)), 0)),
            pl.BlockSpec((_BM, _H5),
                         lambda s, i: (jnp.where(s == 1, i,
                                                 jnp.where(s == 0, 0, last)), 0)),
            pl.BlockSpec((_BM, _H4),
                         lambda s, i: (jnp.where(s == 1, i,
                                                 jnp.where(s == 0, 0, last)), 0)),
            pl.BlockSpec((_BM, _SUMF),
                         lambda s, i: (jnp.where(s == 1, i,
                                                 jnp.where(s == 0, 0, last)), 0)),
            pl.BlockSpec((_BM, _FINC),
                         lambda s, i: (jnp.where(s == 2, i, 0), 0)),
            pl.BlockSpec((_BM, _H5),
                         lambda s, i: (jnp.where(s == 2, i, 0), 0)),
            pl.BlockSpec((_BM, _H4),
                         lambda s, i: (jnp.where(s == 2, i, 0), 0)),
        ],
        out_shape=[
            jax.ShapeDtypeStruct((_N, _SUMF), jnp.float32),
            jax.ShapeDtypeStruct((_N, _FINC), jnp.float32),
            jax.ShapeDtypeStruct((_N, _H5), jnp.float32),
            jax.ShapeDtypeStruct((_N, _H4), jnp.float32),
            jax.ShapeDtypeStruct((_N, _SUMF), jnp.float32),
            jax.ShapeDtypeStruct((_N, _FINC), jnp.float32),
            jax.ShapeDtypeStruct((_N, _H5), jnp.float32),
            jax.ShapeDtypeStruct((_N, _H4), jnp.float32),
        ],
        scratch_shapes=[
            pltpu.VMEM((_N, _SUMF), jnp.bfloat16),
            pltpu.VMEM((_N, _W2C), jnp.bfloat16),
            pltpu.VMEM((_N, _SUMF), jnp.bfloat16),
            pltpu.VMEM((_N, _SUMF), jnp.bfloat16),
            pltpu.VMEM((_N, _N), jnp.bfloat16),
        ],
    )(xbf, ybf, adj1, adj2, wc, bc, w45, b45, wmT, bm2)
    (x_lr, x_final, x_fiv, x_mlp, y_lr, y_final, y_fiv, y_mlp) = outs
    return (x_lr, y_lr, x_final, y_final, x_fiv, x_mlp, y_fiv, y_mlp)


# speedup vs baseline: 1.1834x; 1.1834x over previous
import jax
import jax.numpy as jnp
from jax.experimental import pallas as pl
from jax.experimental.pallas import tpu as pltpu

_N = 4096
_NFEAT = 128
_F0, _F1, _F2 = 64, 68, 72
_SUMF = _F0 + _F1 + _F2
_H4 = _F0 * 2 + 4
_H5 = _F0 * 2
_W2C = _H5 + _H4
_FINC = _SUMF + _H4
_BM = 512
_NB = _N // _BM


def _phase0(adj_ref, s1_ref, bc_ref, lrB_ref, lr_out, row0):
    z = jnp.dot(adj_ref[...].astype(jnp.bfloat16), s1_ref[...],
                preferred_element_type=jnp.float32) + bc_ref[...]
    sig = jax.nn.sigmoid(z)
    lane = jax.lax.broadcasted_iota(jnp.int32, sig.shape, 1)
    sec_mask = (lane >= _F0) & (lane < _F0 + _F1)
    msec = jnp.sum(jnp.where(sec_mask, sig, 0.0), axis=1,
                   keepdims=True) * (1.0 / _F1)
    lr_blk = jnp.where(lane < _F0 + _F1, sig, msec * sig)
    lrB_ref[pl.ds(row0, _BM), :] = lr_blk.astype(jnp.bfloat16)
    lr_out[...] = lr_blk


def _phase1(adj_ref, s2_ref, b45_ref, wm_ref, bm_ref, lrB_ref,
            fin_out, fiv_out, mlp_out, row0):
    z = jnp.dot(adj_ref[...].astype(jnp.bfloat16), s2_ref[...],
                preferred_element_type=jnp.float32) + b45_ref[...]
    fiv = z[:, :_H5]
    fou = z[:, _H5:]
    mlp = jnp.dot(fiv.astype(jnp.bfloat16), wm_ref[...],
                  preferred_element_type=jnp.float32) + bm_ref[...]
    mlp = jnp.where(mlp >= 0.0, mlp, 0.01 * mlp)
    f3 = (mlp + fou) * 0.5
    lr = lrB_ref[pl.ds(row0, _BM), :].astype(jnp.float32)
    low = jnp.mean(lr, axis=1, keepdims=True) * lr + lr
    fin_out[...] = jnp.concatenate([low, f3], axis=1)
    fiv_out[...] = fiv
    mlp_out[...] = mlp


def _body(x_ref, y_ref, adj1_ref, adj2_ref, wc_ref, bc_ref, w45_ref,
          b45_ref, wm_ref, bm_ref,
          xlr_out, xfin_out, xfiv_out, xmlp_out,
          ylr_out, yfin_out, yfiv_out, ymlp_out,
          s1_ref, s2_ref, lrx_ref, lry_ref):
    s = pl.program_id(0)
    i = pl.program_id(1)
    row0 = pl.multiple_of(i * _BM, _BM)

    @pl.when((s == 0) & (i == 0))
    def _():
        s1_ref[...] = jnp.dot(x_ref[...], wc_ref[...],
                              preferred_element_type=jnp.float32
                              ).astype(jnp.bfloat16)

    @pl.when((s == 1) & (i == 0))
    def _():
        s2_ref[...] = jnp.dot(lrx_ref[...], w45_ref[...],
                              preferred_element_type=jnp.float32
                              ).astype(jnp.bfloat16)

    @pl.when((s == 2) & (i == 0))
    def _():
        s1_ref[...] = jnp.dot(y_ref[...], wc_ref[...],
                              preferred_element_type=jnp.float32
                              ).astype(jnp.bfloat16)

    @pl.when((s == 3) & (i == 0))
    def _():
        s2_ref[...] = jnp.dot(lry_ref[...], w45_ref[...],
                              preferred_element_type=jnp.float32
                              ).astype(jnp.bfloat16)

    @pl.when(s == 0)
    def _():
        _phase0(adj1_ref, s1_ref, bc_ref, lrx_ref, xlr_out, row0)

    @pl.when(s == 1)
    def _():
        _phase1(adj1_ref, s2_ref, b45_ref, wm_ref, bm_ref, lrx_ref,
                xfin_out, xfiv_out, xmlp_out, row0)

    @pl.when(s == 2)
    def _():
        _phase0(adj2_ref, s1_ref, bc_ref, lry_ref, ylr_out, row0)

    @pl.when(s == 3)
    def _():
        _phase1(adj2_ref, s2_ref, b45_ref, wm_ref, bm_ref, lry_ref,
                yfin_out, yfiv_out, ymlp_out, row0)


def kernel(x, adj1, y, adj2, W1, b1, W2, b2, W3, b3, W4, b4, W5, b5, Wm, bm):
    wc = jnp.concatenate([W1, W2, W3], axis=1).astype(jnp.bfloat16)
    bc = jnp.concatenate([b1, b2, b3]).reshape(1, _SUMF)
    w45 = jnp.concatenate([W5, W4], axis=1).astype(jnp.bfloat16)
    b45 = jnp.concatenate([b5, b4]).reshape(1, _W2C)
    wmT = Wm.T.astype(jnp.bfloat16)
    bm2 = bm.reshape(1, _H4)
    xbf = x.astype(jnp.bfloat16)
    ybf = y.astype(jnp.bfloat16)

    last = _NB - 1
    outs = pl.pallas_call(
        _body,
        grid=(4, _NB),
        in_specs=[
            pl.BlockSpec((_N, _NFEAT), lambda s, i: (0, 0)),
            pl.BlockSpec((_N, _NFEAT), lambda s, i: (0, 0)),
            pl.BlockSpec((_BM, _N),
                         lambda s, i: (jnp.where(s <= 1, i, last), 0)),
            pl.BlockSpec((_BM, _N),
                         lambda s, i: (jnp.where(s >= 2, i, 0), 0)),
            pl.BlockSpec((_NFEAT, _SUMF), lambda s, i: (0, 0)),
            pl.BlockSpec((1, _SUMF), lambda s, i: (0, 0)),
            pl.BlockSpec((_SUMF, _W2C), lambda s, i: (0, 0)),
            pl.BlockSpec((1, _W2C), lambda s, i: (0, 0)),
            pl.BlockSpec((_H5, _H4), lambda s, i: (0, 0)),
            pl.BlockSpec((1, _H4), lambda s, i: (0, 0)),
        ],
        out_specs=[
            pl.BlockSpec((_BM, _SUMF),
                         lambda s, i: (jnp.where(s == 0, i, last), 0)),
            pl.BlockSpec((_BM, _FINC),
                         lambda s, i: (jnp.where(s == 1, i,
                                                 jnp.where(s == 0, 0, last)), 0)),
            pl.BlockSpec((_BM, _H5),
                         lambda s, i: (jnp.where(s == 1, i,
                                                 jnp.where(s == 0, 0, last)), 0)),
            pl.BlockSpec((_BM, _H4),
                         lambda s, i: (jnp.where(s == 1, i,
                                                 jnp.where(s == 0, 0, last)), 0)),
            pl.BlockSpec((_BM, _SUMF),
                         lambda s, i: (jnp.where(s == 2, i,
                                                 jnp.where(s < 2, 0, last)), 0)),
            pl.BlockSpec((_BM, _FINC),
                         lambda s, i: (jnp.where(s == 3, i, 0), 0)),
            pl.BlockSpec((_BM, _H5),
                         lambda s, i: (jnp.where(s == 3, i, 0), 0)),
            pl.BlockSpec((_BM, _H4),
                         lambda s, i: (jnp.where(s == 3, i, 0), 0)),
        ],
        out_shape=[
            jax.ShapeDtypeStruct((_N, _SUMF), jnp.float32),
            jax.ShapeDtypeStruct((_N, _FINC), jnp.float32),
            jax.ShapeDtypeStruct((_N, _H5), jnp.float32),
            jax.ShapeDtypeStruct((_N, _H4), jnp.float32),
            jax.ShapeDtypeStruct((_N, _SUMF), jnp.float32),
            jax.ShapeDtypeStruct((_N, _FINC), jnp.float32),
            jax.ShapeDtypeStruct((_N, _H5), jnp.float32),
            jax.ShapeDtypeStruct((_N, _H4), jnp.float32),
        ],
        scratch_shapes=[
            pltpu.VMEM((_N, _SUMF), jnp.bfloat16),
            pltpu.VMEM((_N, _W2C), jnp.bfloat16),
            pltpu.VMEM((_N, _SUMF), jnp.bfloat16),
            pltpu.VMEM((_N, _SUMF), jnp.bfloat16),
        ],
    )(xbf, ybf, adj1, adj2, wc, bc, w45, b45, wmT, bm2)
    (x_lr, x_final, x_fiv, x_mlp, y_lr, y_final, y_fiv, y_mlp) = outs
    return (x_lr, y_lr, x_final, y_final, x_fiv, x_mlp, y_fiv, y_mlp)


# speedup vs baseline: 1.2013x; 1.0151x over previous
import jax
import jax.numpy as jnp
from jax.experimental import pallas as pl
from jax.experimental.pallas import tpu as pltpu

_N = 4096
_NFEAT = 128
_F0, _F1, _F2 = 64, 68, 72
_SUMF = _F0 + _F1 + _F2
_H4 = _F0 * 2 + 4
_H5 = _F0 * 2
_W2C = _H5 + _H4
_FINC = _SUMF + _H4
_BM = 1024
_NB = _N // _BM


def _branch_body(x_ref, adj_ref, wc_ref, bc_ref, w45_ref, b45_ref, wm_ref,
                 bm_ref, lr_out, final_out, fiv_out, mlp_out,
                 s1_ref, s2_ref, lr_ref):
    p = pl.program_id(0)
    i = pl.program_id(1)
    row0 = pl.multiple_of(i * _BM, _BM)

    @pl.when((p == 0) & (i == 0))
    def _():
        s1_ref[...] = jnp.dot(x_ref[...], wc_ref[...],
                              preferred_element_type=jnp.float32
                              ).astype(jnp.bfloat16)

    @pl.when(p == 0)
    def _():
        z = jnp.dot(adj_ref[...].astype(jnp.bfloat16), s1_ref[...],
                    preferred_element_type=jnp.float32) + bc_ref[...]
        sig = jax.nn.sigmoid(z)
        lane = jax.lax.broadcasted_iota(jnp.int32, sig.shape, 1)
        sec_mask = (lane >= _F0) & (lane < _F0 + _F1)
        msec = jnp.sum(jnp.where(sec_mask, sig, 0.0), axis=1,
                       keepdims=True) * (1.0 / _F1)
        lr_blk = jnp.where(lane < _F0 + _F1, sig, msec * sig)
        lr_ref[pl.ds(row0, _BM), :] = lr_blk.astype(jnp.bfloat16)
        lr_out[...] = lr_blk

    @pl.when((p == 1) & (i == 0))
    def _():
        s2_ref[...] = jnp.dot(lr_ref[...], w45_ref[...],
                              preferred_element_type=jnp.float32
                              ).astype(jnp.bfloat16)

    @pl.when(p == 1)
    def _():
        z = jnp.dot(adj_ref[...].astype(jnp.bfloat16), s2_ref[...],
                    preferred_element_type=jnp.float32) + b45_ref[...]
        fiv = z[:, :_H5]
        fou = z[:, _H5:]
        mlp = jnp.dot(fiv.astype(jnp.bfloat16), wm_ref[...],
                      preferred_element_type=jnp.float32) + bm_ref[...]
        mlp = jnp.where(mlp >= 0.0, mlp, 0.01 * mlp)
        f3 = (mlp + fou) * 0.5
        lr = lr_ref[pl.ds(row0, _BM), :].astype(jnp.float32)
        low = jnp.mean(lr, axis=1, keepdims=True) * lr + lr
        final_out[...] = jnp.concatenate([low, f3], axis=1)
        fiv_out[...] = fiv
        mlp_out[...] = mlp


def _branch(x, adj, wc, bc, w45, b45, wmT, bm2):
    return pl.pallas_call(
        _branch_body,
        grid=(2, _NB),
        in_specs=[
            pl.BlockSpec((_N, _NFEAT), lambda p, i: (0, 0)),
            pl.BlockSpec((_BM, _N), lambda p, i: (i, 0)),
            pl.BlockSpec((_NFEAT, _SUMF), lambda p, i: (0, 0)),
            pl.BlockSpec((1, _SUMF), lambda p, i: (0, 0)),
            pl.BlockSpec((_SUMF, _W2C), lambda p, i: (0, 0)),
            pl.BlockSpec((1, _W2C), lambda p, i: (0, 0)),
            pl.BlockSpec((_H5, _H4), lambda p, i: (0, 0)),
            pl.BlockSpec((1, _H4), lambda p, i: (0, 0)),
        ],
        out_specs=[
            pl.BlockSpec((_BM, _SUMF),
                         lambda p, i: (jnp.where(p == 0, i, _NB - 1), 0)),
            pl.BlockSpec((_BM, _FINC),
                         lambda p, i: (jnp.where(p == 0, 0, i), 0)),
            pl.BlockSpec((_BM, _H5),
                         lambda p, i: (jnp.where(p == 0, 0, i), 0)),
            pl.BlockSpec((_BM, _H4),
                         lambda p, i: (jnp.where(p == 0, 0, i), 0)),
        ],
        out_shape=[
            jax.ShapeDtypeStruct((_N, _SUMF), jnp.float32),
            jax.ShapeDtypeStruct((_N, _FINC), jnp.float32),
            jax.ShapeDtypeStruct((_N, _H5), jnp.float32),
            jax.ShapeDtypeStruct((_N, _H4), jnp.float32),
        ],
        scratch_shapes=[
            pltpu.VMEM((_N, _SUMF), jnp.bfloat16),
            pltpu.VMEM((_N, _W2C), jnp.bfloat16),
            pltpu.VMEM((_N, _SUMF), jnp.bfloat16),
        ],
    )(x, adj, wc, bc, w45, b45, wmT, bm2)


def kernel(x, adj1, y, adj2, W1, b1, W2, b2, W3, b3, W4, b4, W5, b5, Wm, bm):
    wc = jnp.concatenate([W1, W2, W3], axis=1).astype(jnp.bfloat16)
    bc = jnp.concatenate([b1, b2, b3]).reshape(1, _SUMF)
    w45 = jnp.concatenate([W5, W4], axis=1).astype(jnp.bfloat16)
    b45 = jnp.concatenate([b5, b4]).reshape(1, _W2C)
    wmT = Wm.T.astype(jnp.bfloat16)
    bm2 = bm.reshape(1, _H4)
    xbf = x.astype(jnp.bfloat16)
    ybf = y.astype(jnp.bfloat16)

    x_lr, x_final, x_fiv, x_mlp = _branch(xbf, adj1, wc, bc, w45, b45, wmT, bm2)
    y_lr, y_final, y_fiv, y_mlp = _branch(ybf, adj2, wc, bc, w45, b45, wmT, bm2)
    return (x_lr, y_lr, x_final, y_final, x_fiv, x_mlp, y_fiv, y_mlp)
